# Initial kernel scaffold; baseline (speedup 1.0000x reference)
import functools

import jax
import jax.numpy as jnp
from jax import lax
from jax.experimental import pallas as pl
from jax.experimental.pallas import tpu as pltpu
from jax.experimental.pallas import tpu_sc as plsc

N = 10000
NP = 10240
E = 320000
D = 128
DP = 144
NTILES = 32
EPT = E // NTILES
C = 80
NCH = EPT // C
RPT = NP // 16
LANES = 16


def _tc_project(xp, W, attn_l, attn_r):
    def body(x_ref, w_ref, al_ref, ar_ref, featx_ref, el_ref, er_ref):
        feat = jnp.dot(x_ref[...], w_ref[...],
                       preferred_element_type=jnp.float32)
        col = lax.broadcasted_iota(jnp.int32, (NP, DP - D), 1)
        featx_ref[:, :D] = feat
        featx_ref[:, D:] = jnp.where(col == 0, 1.0, 0.0).astype(jnp.float32)
        el_ref[...] = jnp.sum(feat * al_ref[...][None, :], axis=1)
        er_ref[...] = jnp.sum(feat * ar_ref[...][None, :], axis=1)

    return pl.pallas_call(
        body,
        out_shape=(
            jax.ShapeDtypeStruct((NP, DP), jnp.float32),
            jax.ShapeDtypeStruct((NP,), jnp.float32),
            jax.ShapeDtypeStruct((NP,), jnp.float32),
        ),
    )(xp, W, attn_l, attn_r)


def _sc_edge_aggregate(featx, el, er, src3, dst3, zeros):
    mesh = plsc.VectorSubcoreMesh(core_axis_name="c", subcore_axis_name="s")

    @functools.partial(
        pl.kernel,
        out_type=jax.ShapeDtypeStruct((2, NP, DP), jnp.float32),
        mesh=mesh,
        scratch_types=[
            pltpu.VMEM((NP,), jnp.float32),
            pltpu.VMEM((NP,), jnp.float32),
            pltpu.VMEM((NCH, C), jnp.int32),
            pltpu.VMEM((NCH, C), jnp.int32),
            pltpu.VMEM((C, DP), jnp.float32),
            pltpu.VMEM((C,), jnp.float32),
            pltpu.VMEM_SHARED((NP, DP), jnp.float32),
            pltpu.SemaphoreType.DMA,
        ],
    )
    def k(featx_hbm, el_hbm, er_hbm, src_hbm, dst_hbm, zeros_hbm, out_hbm,
          el_v, er_v, src_v, dst_v, rows_v, w_v, acc_sh, sem):
        cid = lax.axis_index("c")
        sid = lax.axis_index("s")
        wid = cid * 16 + sid

        pltpu.sync_copy(zeros_hbm.at[pl.ds(sid * RPT, RPT)],
                        acc_sh.at[pl.ds(sid * RPT, RPT)])
        pltpu.sync_copy(el_hbm, el_v)
        pltpu.sync_copy(er_hbm, er_v)
        pltpu.sync_copy(src_hbm.at[wid], src_v)
        pltpu.sync_copy(dst_hbm.at[wid], dst_v)
        plsc.subcore_barrier()

        @pl.loop(0, NCH)
        def _chunk(j):
            pltpu.async_copy(featx_hbm.at[src_v.at[j]], rows_v, sem).wait()

            for k0 in range(0, C, LANES):
                si = src_v[j, pl.ds(k0, LANES)]
                di = dst_v[j, pl.ds(k0, LANES)]
                e = plsc.load_gather(el_v, [si]) + plsc.load_gather(er_v, [di])
                e = jnp.where(e >= 0.0, e, 0.2 * e)
                w_v[pl.ds(k0, LANES)] = jnp.exp(e)

            @pl.loop(0, C)
            def _row(r):
                wb = jnp.full((LANES,), w_v[r], dtype=jnp.float32)
                for q in range(DP // LANES):
                    sl = pl.ds(q * LANES, LANES)
                    rows_v[r, sl] = rows_v[r, sl] * wb

            pltpu.sync_copy(rows_v, acc_sh.at[dst_v.at[j]], add=True)

        plsc.subcore_barrier()
        pltpu.sync_copy(acc_sh.at[pl.ds(sid * RPT, RPT)],
                        out_hbm.at[cid, pl.ds(sid * RPT, RPT)])

    return k(featx, el, er, src3, dst3, zeros)


def _tc_finalize(partials, bias):
    def body(p_ref, b_ref, o_ref):
        s = p_ref[0] + p_ref[1]
        num = s[:, :D]
        den = s[:, D:D + 1]
        o_ref[...] = jnp.where(den > 0.0, num / den, 0.0) + b_ref[...][None, :]

    return pl.pallas_call(
        body,
        out_shape=jax.ShapeDtypeStruct((NP, D), jnp.float32),
    )(partials, bias)


def kernel(x, edge_index, W, attn_l, attn_r, bias):
    xp = jnp.zeros((NP, D), jnp.float32).at[:N].set(x)
    featx, el, er = _tc_project(xp, W, attn_l, attn_r)
    src3 = edge_index[0].reshape(NTILES, NCH, C)
    dst3 = edge_index[1].reshape(NTILES, NCH, C)
    zeros = jnp.zeros((NP, DP), jnp.float32)
    partials = _sc_edge_aggregate(featx, el, er, src3, dst3, zeros)
    return _tc_finalize(partials, bias)[:N]


# speedup vs baseline: 18.7184x; 18.7184x over previous
import dataclasses
import functools

import jax
import jax.numpy as jnp
from jax import lax
from jax.experimental import pallas as pl
from jax.experimental.pallas import tpu as pltpu
from jax.experimental.pallas import tpu_sc as plsc

N = 10000
NP = 10240
E = 320000
D = 128
DH = 64
DP = 80
NSUB = 16
EPT = E // NSUB
C = 80
NCH = EPT // C
RPT = NP // NSUB
LANES = 16


def _tc_project(xp, W, attn_l, attn_r):
    def body(x_ref, w_ref, al_ref, ar_ref, feat2_ref, el_ref, er_ref):
        feat = jnp.dot(x_ref[...], w_ref[...],
                       preferred_element_type=jnp.float32)
        col = lax.broadcasted_iota(jnp.int32, (NP, DP - DH), 1)
        pad = jnp.where(col == 0, 1.0, 0.0).astype(jnp.float32)
        feat2_ref[0, :, :DH] = feat[:, :DH]
        feat2_ref[0, :, DH:] = pad
        feat2_ref[1, :, :DH] = feat[:, DH:]
        feat2_ref[1, :, DH:] = pad
        el_ref[...] = jnp.sum(feat * al_ref[...][None, :], axis=1)
        er_ref[...] = jnp.sum(feat * ar_ref[...][None, :], axis=1)

    return pl.pallas_call(
        body,
        out_shape=(
            jax.ShapeDtypeStruct((2, NP, DP), jnp.float32),
            jax.ShapeDtypeStruct((NP,), jnp.float32),
            jax.ShapeDtypeStruct((NP,), jnp.float32),
        ),
    )(xp, W, attn_l, attn_r)


def _sc_edge_aggregate(feat2, el, er, src3, dst3, zeros):
    mesh = plsc.VectorSubcoreMesh(core_axis_name="c", subcore_axis_name="s")
    cp = pltpu.CompilerParams()
    if "needs_layout_passes" in pltpu.CompilerParams.__dataclass_fields__:
        cp = dataclasses.replace(cp, needs_layout_passes=False)
    if "use_tc_tiling_on_sc" in pltpu.CompilerParams.__dataclass_fields__:
        cp = dataclasses.replace(cp, use_tc_tiling_on_sc=False)

    @functools.partial(
        pl.kernel,
        compiler_params=cp,
        out_type=jax.ShapeDtypeStruct((2, NP, DP), jnp.float32),
        mesh=mesh,
        scratch_types=[
            pltpu.VMEM((NP,), jnp.float32),
            pltpu.VMEM((NP,), jnp.float32),
            pltpu.VMEM((NCH, C), jnp.int32),
            pltpu.VMEM((NCH, C), jnp.int32),
            pltpu.VMEM((C, DP), jnp.float32),
            pltpu.VMEM((C,), jnp.float32),
            pltpu.VMEM_SHARED((NP, DP), jnp.float32),
            pltpu.SemaphoreType.DMA,
        ],
    )
    def k(feat2_hbm, el_hbm, er_hbm, src_hbm, dst_hbm, zeros_hbm, out_hbm,
          el_v, er_v, src_v, dst_v, rows_v, w_v, acc_sh, sem):
        cid = lax.axis_index("c")
        sid = lax.axis_index("s")

        pltpu.sync_copy(zeros_hbm.at[pl.ds(sid * RPT, RPT)],
                        acc_sh.at[pl.ds(sid * RPT, RPT)])
        pltpu.sync_copy(el_hbm, el_v)
        pltpu.sync_copy(er_hbm, er_v)
        pltpu.sync_copy(src_hbm.at[sid], src_v)
        pltpu.sync_copy(dst_hbm.at[sid], dst_v)
        plsc.subcore_barrier()

        @pl.loop(0, NCH)
        def _chunk(j):
            pltpu.async_copy(feat2_hbm.at[cid].at[src_v.at[j]],
                             rows_v, sem).wait()

            for k0 in range(0, C, LANES):
                si = src_v[j, pl.ds(k0, LANES)]
                di = dst_v[j, pl.ds(k0, LANES)]
                e = plsc.load_gather(el_v, [si]) + plsc.load_gather(er_v, [di])
                e = jnp.where(e >= 0.0, e, 0.2 * e)
                w_v[pl.ds(k0, LANES)] = jnp.exp(e)

            @pl.loop(0, C)
            def _row(r):
                ridx = jnp.full((LANES,), r, dtype=jnp.int32)
                wb = plsc.load_gather(w_v, [ridx])
                for q in range(DP // LANES):
                    sl = pl.ds(q * LANES, LANES)
                    rows_v[r, sl] = rows_v[r, sl] * wb

            pltpu.sync_copy(rows_v, acc_sh.at[dst_v.at[j]], add=True)

        plsc.subcore_barrier()
        pltpu.sync_copy(acc_sh.at[pl.ds(sid * RPT, RPT)],
                        out_hbm.at[cid, pl.ds(sid * RPT, RPT)])

    return k(feat2, el, er, src3, dst3, zeros)


def _tc_finalize(partials, bias):
    def body(p_ref, b_ref, o_ref):
        num = jnp.concatenate([p_ref[0, :, :DH], p_ref[1, :, :DH]], axis=1)
        den = p_ref[0, :, DH:DH + 1]
        o_ref[...] = jnp.where(den > 0.0, num / den, 0.0) + b_ref[...][None, :]

    return pl.pallas_call(
        body,
        out_shape=jax.ShapeDtypeStruct((NP, D), jnp.float32),
    )(partials, bias)


def kernel(x, edge_index, W, attn_l, attn_r, bias):
    xp = jnp.zeros((NP, D), jnp.float32).at[:N].set(x)
    feat2, el, er = _tc_project(xp, W, attn_l, attn_r)
    src3 = edge_index[0].reshape(NSUB, NCH, C)
    dst3 = edge_index[1].reshape(NSUB, NCH, C)
    zeros = jnp.zeros((NP, DP), jnp.float32)
    partials = _sc_edge_aggregate(feat2, el, er, src3, dst3, zeros)
    return _tc_finalize(partials, bias)[:N]


# speedup vs baseline: 31.5876x; 1.6875x over previous
import dataclasses
import functools

import jax
import jax.numpy as jnp
from jax import lax
from jax.experimental import pallas as pl
from jax.experimental.pallas import tpu as pltpu
from jax.experimental.pallas import tpu_sc as plsc

N = 10000
NP = 10240
E = 320000
D = 128
DH = 64
DP = 80
NSUB = 16
EPT = E // NSUB
C = 80
NCH = EPT // C
RPT = NP // NSUB
LANES = 16
NBUF = 4


def _tc_project(xp, W, attn_l, attn_r):
    def body(x_ref, w_ref, al_ref, ar_ref, feat2_ref, el_ref, er_ref):
        feat = jnp.dot(x_ref[...], w_ref[...],
                       preferred_element_type=jnp.float32)
        col = lax.broadcasted_iota(jnp.int32, (NP, DP - DH), 1)
        pad = jnp.where(col == 0, 1.0, 0.0).astype(jnp.float32)
        feat2_ref[0, :, :DH] = feat[:, :DH]
        feat2_ref[0, :, DH:] = pad
        feat2_ref[1, :, :DH] = feat[:, DH:]
        feat2_ref[1, :, DH:] = pad
        el_ref[...] = jnp.sum(feat * al_ref[...][None, :], axis=1)
        er_ref[...] = jnp.sum(feat * ar_ref[...][None, :], axis=1)

    return pl.pallas_call(
        body,
        out_shape=(
            jax.ShapeDtypeStruct((2, NP, DP), jnp.float32),
            jax.ShapeDtypeStruct((NP,), jnp.float32),
            jax.ShapeDtypeStruct((NP,), jnp.float32),
        ),
    )(xp, W, attn_l, attn_r)


def _sc_edge_aggregate(feat2, el, er, idx4, zeros):
    mesh = plsc.VectorSubcoreMesh(core_axis_name="c", subcore_axis_name="s")
    cp = pltpu.CompilerParams()
    if "needs_layout_passes" in pltpu.CompilerParams.__dataclass_fields__:
        cp = dataclasses.replace(cp, needs_layout_passes=False)
    if "use_tc_tiling_on_sc" in pltpu.CompilerParams.__dataclass_fields__:
        cp = dataclasses.replace(cp, use_tc_tiling_on_sc=False)

    @functools.partial(
        pl.kernel,
        compiler_params=cp,
        out_type=jax.ShapeDtypeStruct((2, NP, DP), jnp.float32),
        mesh=mesh,
        scratch_types=(
            [
                pltpu.VMEM((NP,), jnp.float32),
                pltpu.VMEM((NP,), jnp.float32),
            ]
            + [pltpu.VMEM((2, C), jnp.int32)] * NBUF
            + [pltpu.VMEM((C, DP), jnp.float32)] * NBUF
            + [
                pltpu.VMEM((C,), jnp.float32),
                pltpu.VMEM_SHARED((NP, DP), jnp.float32),
            ]
            + [pltpu.SemaphoreType.DMA] * (3 * NBUF + 1)
        ),
    )
    def k(feat2_hbm, el_hbm, er_hbm, idx_hbm, zeros_hbm, out_hbm,
          el_v, er_v, i0, i1, i2, i3, r0, r1, r2, r3, w_v, acc_sh,
          a0, a1, a2, a3, g0, g1, g2, g3, s0, s1, s2, s3, stg):
        cid = lax.axis_index("c")
        sid = lax.axis_index("s")
        idxb = [i0, i1, i2, i3]
        rows = [r0, r1, r2, r3]
        isem = [a0, a1, a2, a3]
        gsem = [g0, g1, g2, g3]
        ssem = [s0, s1, s2, s3]

        copies = [
            (zeros_hbm.at[pl.ds(sid * RPT, RPT)],
             acc_sh.at[pl.ds(sid * RPT, RPT)]),
            (el_hbm, el_v),
            (er_hbm, er_v),
        ]
        descs = [pltpu.async_copy(a, b, stg) for a, b in copies]
        for d in descs:
            d.wait()
        plsc.subcore_barrier()

        def idx_start(m, b):
            pltpu.async_copy(idx_hbm.at[sid, m], idxb[b], isem[b])

        def idx_wait(m, b):
            pltpu.make_async_copy(idx_hbm.at[sid, m], idxb[b], isem[b]).wait()

        def gather_start(b):
            pltpu.async_copy(feat2_hbm.at[cid].at[idxb[b].at[0]],
                             rows[b], gsem[b])

        def gather_wait(b):
            pltpu.make_async_copy(feat2_hbm.at[cid].at[idxb[b].at[0]],
                                  rows[b], gsem[b]).wait()

        def scat_start(b):
            pltpu.async_copy(rows[b], acc_sh.at[idxb[b].at[1]], ssem[b],
                             add=True)

        def scat_wait(b):
            pltpu.make_async_copy(rows[b], acc_sh.at[idxb[b].at[1]],
                                  ssem[b]).wait()

        def process(b):
            for k0 in range(0, C, LANES):
                si = idxb[b][0, pl.ds(k0, LANES)]
                di = idxb[b][1, pl.ds(k0, LANES)]
                e = plsc.load_gather(el_v, [si]) + plsc.load_gather(er_v, [di])
                e = jnp.where(e >= 0.0, e, 0.2 * e)
                w_v[pl.ds(k0, LANES)] = jnp.exp(e)

            @pl.loop(0, C, step=2)
            def _row(r):
                for rr in range(2):
                    ridx = jnp.full((LANES,), r + rr, dtype=jnp.int32)
                    wb = plsc.load_gather(w_v, [ridx])
                    for q in range(DP // LANES):
                        sl = pl.ds(q * LANES, LANES)
                        rows[b][r + rr, sl] = rows[b][r + rr, sl] * wb

        idx_start(0, 0)
        idx_start(1, 1)
        idx_wait(0, 0)
        gather_start(0)
        idx_start(2, 2)
        idx_wait(1, 1)
        gather_start(1)
        gather_wait(0)
        process(0)
        scat_start(0)

        idx_start(3, 3)
        idx_wait(2, 2)
        gather_start(2)
        gather_wait(1)
        process(1)
        scat_start(1)

        scat_wait(0)
        idx_start(4, 0)
        idx_wait(3, 3)
        gather_start(3)
        gather_wait(2)
        process(2)
        scat_start(2)

        scat_wait(1)
        idx_start(5, 1)
        idx_wait(4, 0)
        gather_start(0)
        gather_wait(3)
        process(3)
        scat_start(3)

        @pl.loop(4, NCH - 2, step=NBUF)
        def _steady(j):
            for b in range(NBUF):
                m = j + b
                bb = (b + 2) % NBUF
                b1 = (b + 1) % NBUF
                scat_wait(bb)
                idx_start(m + 2, bb)
                idx_wait(m + 1, b1)
                gather_start(b1)
                gather_wait(b)
                process(b)
                scat_start(b)

        scat_wait(2)
        idx_wait(NCH - 1, 1)
        gather_start(1)
        gather_wait(0)
        process(0)
        scat_start(0)

        scat_wait(3)
        gather_wait(1)
        process(1)
        scat_start(1)
        scat_wait(0)
        scat_wait(1)

        plsc.subcore_barrier()
        pltpu.sync_copy(acc_sh.at[pl.ds(sid * RPT, RPT)],
                        out_hbm.at[cid, pl.ds(sid * RPT, RPT)])

    return k(feat2, el, er, idx4, zeros)


def _tc_finalize(partials, bias):
    def body(p_ref, b_ref, o_ref):
        num = jnp.concatenate([p_ref[0, :, :DH], p_ref[1, :, :DH]], axis=1)
        den = p_ref[0, :, DH:DH + 1]
        o_ref[...] = jnp.where(den > 0.0, num / den, 0.0) + b_ref[...][None, :]

    return pl.pallas_call(
        body,
        out_shape=jax.ShapeDtypeStruct((NP, D), jnp.float32),
    )(partials, bias)


def kernel(x, edge_index, W, attn_l, attn_r, bias):
    xp = jnp.zeros((NP, D), jnp.float32).at[:N].set(x)
    feat2, el, er = _tc_project(xp, W, attn_l, attn_r)
    src3 = edge_index[0].reshape(NSUB, NCH, C)
    dst3 = edge_index[1].reshape(NSUB, NCH, C)
    idx4 = jnp.stack([src3, dst3], axis=2)
    zeros = jnp.zeros((NP, DP), jnp.float32)
    partials = _sc_edge_aggregate(feat2, el, er, idx4, zeros)
    return _tc_finalize(partials, bias)[:N]


# speedup vs baseline: 32.3829x; 1.0252x over previous
import dataclasses
import functools

import jax
import jax.numpy as jnp
from jax import lax
from jax.experimental import pallas as pl
from jax.experimental.pallas import tpu as pltpu
from jax.experimental.pallas import tpu_sc as plsc

N = 10000
NP = 10240
E = 320000
D = 128
DG = 64
DDEN = 16
NSUB = 16
EPT = E // NSUB
C = 80
NCH = EPT // C
RPT = NP // NSUB
LANES = 16
NBUF = 4


def _tc_project(x, W, attn_l, attn_r):
    def body(x_ref, w_ref, al_ref, ar_ref, feat2_ref, el_ref, er_ref):
        feat = jnp.dot(x_ref[...], w_ref[...],
                       preferred_element_type=jnp.float32)
        feat2_ref[0, :N, :] = feat[:, :DG]
        feat2_ref[1, :N, :] = feat[:, DG:]
        el_ref[pl.ds(0, N)] = jnp.sum(feat * al_ref[...][None, :], axis=1)
        er_ref[pl.ds(0, N)] = jnp.sum(feat * ar_ref[...][None, :], axis=1)

    return pl.pallas_call(
        body,
        out_shape=(
            jax.ShapeDtypeStruct((2, NP, DG), jnp.float32),
            jax.ShapeDtypeStruct((NP,), jnp.float32),
            jax.ShapeDtypeStruct((NP,), jnp.float32),
        ),
    )(x, W, attn_l, attn_r)


def _sc_edge_aggregate(feat2, el, er, idx4, z64, z16):
    mesh = plsc.VectorSubcoreMesh(core_axis_name="c", subcore_axis_name="s")
    cp = pltpu.CompilerParams()
    if "needs_layout_passes" in pltpu.CompilerParams.__dataclass_fields__:
        cp = dataclasses.replace(cp, needs_layout_passes=False)
    if "use_tc_tiling_on_sc" in pltpu.CompilerParams.__dataclass_fields__:
        cp = dataclasses.replace(cp, use_tc_tiling_on_sc=False)

    @functools.partial(
        pl.kernel,
        compiler_params=cp,
        out_type=(
            jax.ShapeDtypeStruct((2, NP, DG), jnp.float32),
            jax.ShapeDtypeStruct((2, NP, DDEN), jnp.float32),
        ),
        mesh=mesh,
        scratch_types=(
            [
                pltpu.VMEM((NP,), jnp.float32),
                pltpu.VMEM((NP,), jnp.float32),
            ]
            + [pltpu.VMEM((2, C), jnp.int32)] * NBUF
            + [pltpu.VMEM((C, DG), jnp.float32)] * NBUF
            + [pltpu.VMEM((C, DDEN), jnp.float32)] * NBUF
            + [
                pltpu.VMEM_SHARED((NP, DG), jnp.float32),
                pltpu.VMEM_SHARED((NP, DDEN), jnp.float32),
            ]
            + [pltpu.SemaphoreType.DMA] * (3 * NBUF + 1)
        ),
    )
    def k(feat2_hbm, el_hbm, er_hbm, idx_hbm, z64_hbm, z16_hbm,
          acc_out, den_out,
          el_v, er_v, i0, i1, i2, i3, r0, r1, r2, r3, w0, w1, w2, w3,
          acc_sh, den_sh,
          a0, a1, a2, a3, g0, g1, g2, g3, s0, s1, s2, s3, stg):
        cid = lax.axis_index("c")
        sid = lax.axis_index("s")
        idxb = [i0, i1, i2, i3]
        rows = [r0, r1, r2, r3]
        wden = [w0, w1, w2, w3]
        isem = [a0, a1, a2, a3]
        gsem = [g0, g1, g2, g3]
        ssem = [s0, s1, s2, s3]
        tile = pl.ds(sid * RPT, RPT)

        copies = [
            (z64_hbm.at[tile], acc_sh.at[tile]),
            (z16_hbm.at[tile], den_sh.at[tile]),
            (el_hbm, el_v),
            (er_hbm, er_v),
        ]
        descs = [pltpu.async_copy(a, b, stg) for a, b in copies]
        for d in descs:
            d.wait()
        plsc.subcore_barrier()

        def idx_start(m, b):
            pltpu.async_copy(idx_hbm.at[sid, m], idxb[b], isem[b])

        def idx_wait(m, b):
            pltpu.make_async_copy(idx_hbm.at[sid, m], idxb[b], isem[b]).wait()

        def gather_start(b):
            pltpu.async_copy(feat2_hbm.at[cid].at[idxb[b].at[0]],
                             rows[b], gsem[b])

        def gather_wait(b):
            pltpu.make_async_copy(feat2_hbm.at[cid].at[idxb[b].at[0]],
                                  rows[b], gsem[b]).wait()

        def scat_start(b):
            pltpu.async_copy(rows[b], acc_sh.at[idxb[b].at[1]], ssem[b],
                             add=True)
            pltpu.async_copy(wden[b], den_sh.at[idxb[b].at[1]], ssem[b],
                             add=True)

        def scat_wait(b):
            pltpu.make_async_copy(rows[b], acc_sh.at[idxb[b].at[1]],
                                  ssem[b]).wait()
            pltpu.make_async_copy(wden[b], den_sh.at[idxb[b].at[1]],
                                  ssem[b]).wait()

        def process(b):
            @pl.loop(0, C, step=LANES)
            def _grp(g):
                si = idxb[b][0, pl.ds(g, LANES)]
                di = idxb[b][1, pl.ds(g, LANES)]
                e = plsc.load_gather(el_v, [si]) + plsc.load_gather(er_v, [di])
                e = jnp.where(e >= 0.0, e, 0.2 * e)
                wv = jnp.exp(e)
                for rr in range(LANES):
                    wb = jnp.broadcast_to(wv[rr], (LANES,))
                    wden[b][g + rr, :] = wb
                    for q in range(DG // LANES):
                        sl = pl.ds(q * LANES, LANES)
                        rows[b][g + rr, sl] = rows[b][g + rr, sl] * wb

        idx_start(0, 0)
        idx_start(1, 1)
        idx_wait(0, 0)
        gather_start(0)
        idx_start(2, 2)
        idx_wait(1, 1)
        gather_start(1)
        gather_wait(0)
        process(0)
        scat_start(0)

        idx_start(3, 3)
        idx_wait(2, 2)
        gather_start(2)
        gather_wait(1)
        process(1)
        scat_start(1)

        scat_wait(0)
        idx_start(4, 0)
        idx_wait(3, 3)
        gather_start(3)
        gather_wait(2)
        process(2)
        scat_start(2)

        scat_wait(1)
        idx_start(5, 1)
        idx_wait(4, 0)
        gather_start(0)
        gather_wait(3)
        process(3)
        scat_start(3)

        @pl.loop(4, NCH - 2, step=NBUF)
        def _steady(j):
            for b in range(NBUF):
                m = j + b
                bb = (b + 2) % NBUF
                b1 = (b + 1) % NBUF
                scat_wait(bb)
                idx_start(m + 2, bb)
                idx_wait(m + 1, b1)
                gather_start(b1)
                gather_wait(b)
                process(b)
                scat_start(b)

        scat_wait(2)
        idx_wait(NCH - 1, 1)
        gather_start(1)
        gather_wait(0)
        process(0)
        scat_start(0)

        scat_wait(3)
        gather_wait(1)
        process(1)
        scat_start(1)
        scat_wait(0)
        scat_wait(1)

        plsc.subcore_barrier()
        pltpu.sync_copy(acc_sh.at[tile], acc_out.at[cid, tile])
        pltpu.sync_copy(den_sh.at[tile], den_out.at[cid, tile])

    return k(feat2, el, er, idx4, z64, z16)


def _tc_finalize(acc, den, bias):
    def body(a_ref, d_ref, b_ref, o_ref):
        num = jnp.concatenate([a_ref[0, :N, :], a_ref[1, :N, :]], axis=1)
        d = d_ref[0, :N, 0:1]
        o_ref[...] = jnp.where(d > 0.0, num / d, 0.0) + b_ref[...][None, :]

    return pl.pallas_call(
        body,
        out_shape=jax.ShapeDtypeStruct((N, D), jnp.float32),
    )(acc, den, bias)


def kernel(x, edge_index, W, attn_l, attn_r, bias):
    feat2, el, er = _tc_project(x, W, attn_l, attn_r)
    src3 = edge_index[0].reshape(NSUB, NCH, C)
    dst3 = edge_index[1].reshape(NSUB, NCH, C)
    idx4 = jnp.stack([src3, dst3], axis=2)
    z64 = jnp.zeros((NP, DG), jnp.float32)
    z16 = jnp.zeros((NP, DDEN), jnp.float32)
    acc, den = _sc_edge_aggregate(feat2, el, er, idx4, z64, z16)
    return _tc_finalize(acc, den, bias)


# speedup vs baseline: 34.0321x; 1.0509x over previous
import dataclasses
import functools

import jax
import jax.numpy as jnp
from jax import lax
from jax.experimental import pallas as pl
from jax.experimental.pallas import tpu as pltpu
from jax.experimental.pallas import tpu_sc as plsc

N = 10000
NP = 10240
E = 320000
D = 128
DG = 64
DDEN = 16
NSUB = 16
EPT = E // NSUB
C = 80
NCH = EPT // C
RPT = NP // NSUB
LANES = 16
NBUF = 4


def _tc_project(x, W, attn_l, attn_r):
    def body(x_ref, w_ref, al_ref, ar_ref, feat2_ref, el_ref, er_ref):
        feat = jnp.dot(x_ref[...], w_ref[...],
                       preferred_element_type=jnp.float32)
        feat2_ref[0, :N, :] = feat[:, :DG]
        feat2_ref[1, :N, :] = feat[:, DG:]
        el_ref[pl.ds(0, N)] = jnp.sum(feat * al_ref[...][None, :], axis=1)
        er_ref[pl.ds(0, N)] = jnp.sum(feat * ar_ref[...][None, :], axis=1)

    return pl.pallas_call(
        body,
        out_shape=(
            jax.ShapeDtypeStruct((2, NP, DG), jnp.float32),
            jax.ShapeDtypeStruct((NP,), jnp.float32),
            jax.ShapeDtypeStruct((NP,), jnp.float32),
        ),
    )(x, W, attn_l, attn_r)


def _sc_edge_aggregate(feat2, el, er, idx4, z64, z16):
    mesh = plsc.VectorSubcoreMesh(core_axis_name="c", subcore_axis_name="s")
    cp = pltpu.CompilerParams()
    if "needs_layout_passes" in pltpu.CompilerParams.__dataclass_fields__:
        cp = dataclasses.replace(cp, needs_layout_passes=False)
    if "use_tc_tiling_on_sc" in pltpu.CompilerParams.__dataclass_fields__:
        cp = dataclasses.replace(cp, use_tc_tiling_on_sc=False)

    @functools.partial(
        pl.kernel,
        compiler_params=cp,
        out_type=(
            jax.ShapeDtypeStruct((2, NP, DG), jnp.float32),
            jax.ShapeDtypeStruct((2, NP, DDEN), jnp.float32),
        ),
        mesh=mesh,
        scratch_types=(
            [
                pltpu.VMEM((NP,), jnp.float32),
                pltpu.VMEM((NP,), jnp.float32),
            ]
            + [pltpu.VMEM((2, C), jnp.int32)] * NBUF
            + [pltpu.VMEM((C, DG), jnp.float32)] * NBUF
            + [pltpu.VMEM((C, DDEN), jnp.float32)] * NBUF
            + [
                pltpu.VMEM_SHARED((NP, DG), jnp.float32),
                pltpu.VMEM_SHARED((NP, DDEN), jnp.float32),
            ]
            + [pltpu.SemaphoreType.DMA] * (3 * NBUF + 1)
        ),
    )
    def k(feat2_hbm, el_hbm, er_hbm, idx_hbm, z64_hbm, z16_hbm,
          acc_out, den_out,
          el_v, er_v, i0, i1, i2, i3, r0, r1, r2, r3, w0, w1, w2, w3,
          acc_sh, den_sh,
          a0, a1, a2, a3, g0, g1, g2, g3, s0, s1, s2, s3, stg):
        cid = lax.axis_index("c")
        sid = lax.axis_index("s")
        idxb = [i0, i1, i2, i3]
        rows = [r0, r1, r2, r3]
        wden = [w0, w1, w2, w3]
        isem = [a0, a1, a2, a3]
        gsem = [g0, g1, g2, g3]
        ssem = [s0, s1, s2, s3]
        tile = pl.ds(sid * RPT, RPT)

        copies = [
            (z64_hbm.at[tile], acc_sh.at[tile]),
            (z16_hbm.at[tile], den_sh.at[tile]),
            (el_hbm, el_v),
            (er_hbm, er_v),
        ]
        descs = [pltpu.async_copy(a, b, stg) for a, b in copies]
        for d in descs:
            d.wait()
        plsc.subcore_barrier()

        def idx_start(m, b):
            pltpu.async_copy(idx_hbm.at[sid, m], idxb[b], isem[b])

        def idx_wait(m, b):
            pltpu.make_async_copy(idx_hbm.at[sid, m], idxb[b], isem[b]).wait()

        def gather_start(b):
            pltpu.async_copy(feat2_hbm.at[cid].at[idxb[b].at[0]],
                             rows[b], gsem[b])

        def gather_wait(b):
            pltpu.make_async_copy(feat2_hbm.at[cid].at[idxb[b].at[0]],
                                  rows[b], gsem[b]).wait()

        def scat_start(b):
            pltpu.async_copy(rows[b], acc_sh.at[idxb[b].at[1]], ssem[b],
                             add=True)
            pltpu.async_copy(wden[b], den_sh.at[idxb[b].at[1]], ssem[b],
                             add=True)

        def scat_wait(b):
            pltpu.make_async_copy(rows[b], acc_sh.at[idxb[b].at[1]],
                                  ssem[b]).wait()
            pltpu.make_async_copy(wden[b], den_sh.at[idxb[b].at[1]],
                                  ssem[b]).wait()

        def process(b):
            @plsc.parallel_loop(0, C, step=LANES)
            def _grp(g):
                si = idxb[b][0, pl.ds(g, LANES)]
                di = idxb[b][1, pl.ds(g, LANES)]
                e = plsc.load_gather(el_v, [si]) + plsc.load_gather(er_v, [di])
                e = jnp.where(e >= 0.0, e, 0.2 * e)
                wv = jnp.exp(e)
                for rr in range(LANES):
                    wb = jnp.broadcast_to(wv[rr], (LANES,))
                    wden[b][g + rr, :] = wb
                    for q in range(DG // LANES):
                        sl = pl.ds(q * LANES, LANES)
                        rows[b][g + rr, sl] = rows[b][g + rr, sl] * wb

        idx_start(0, 0)
        idx_start(1, 1)
        idx_wait(0, 0)
        gather_start(0)
        idx_start(2, 2)
        idx_wait(1, 1)
        gather_start(1)
        gather_wait(0)
        process(0)
        scat_start(0)

        idx_start(3, 3)
        idx_wait(2, 2)
        gather_start(2)
        gather_wait(1)
        process(1)
        scat_start(1)

        scat_wait(0)
        idx_start(4, 0)
        idx_wait(3, 3)
        gather_start(3)
        gather_wait(2)
        process(2)
        scat_start(2)

        scat_wait(1)
        idx_start(5, 1)
        idx_wait(4, 0)
        gather_start(0)
        gather_wait(3)
        process(3)
        scat_start(3)

        @pl.loop(4, NCH - 2, step=NBUF)
        def _steady(j):
            for b in range(NBUF):
                m = j + b
                bb = (b + 2) % NBUF
                b1 = (b + 1) % NBUF
                scat_wait(bb)
                idx_start(m + 2, bb)
                idx_wait(m + 1, b1)
                gather_start(b1)
                gather_wait(b)
                process(b)
                scat_start(b)

        scat_wait(2)
        idx_wait(NCH - 1, 1)
        gather_start(1)
        gather_wait(0)
        process(0)
        scat_start(0)

        scat_wait(3)
        gather_wait(1)
        process(1)
        scat_start(1)
        scat_wait(0)
        scat_wait(1)

        plsc.subcore_barrier()
        pltpu.sync_copy(acc_sh.at[tile], acc_out.at[cid, tile])
        pltpu.sync_copy(den_sh.at[tile], den_out.at[cid, tile])

    return k(feat2, el, er, idx4, z64, z16)


def _tc_finalize(acc, den, bias):
    def body(a_ref, d_ref, b_ref, o_ref):
        num = jnp.concatenate([a_ref[0, :N, :], a_ref[1, :N, :]], axis=1)
        d = d_ref[0, :N, 0:1]
        o_ref[...] = jnp.where(d > 0.0, num / d, 0.0) + b_ref[...][None, :]

    return pl.pallas_call(
        body,
        out_shape=jax.ShapeDtypeStruct((N, D), jnp.float32),
    )(acc, den, bias)


def kernel(x, edge_index, W, attn_l, attn_r, bias):
    feat2, el, er = _tc_project(x, W, attn_l, attn_r)
    src3 = edge_index[0].reshape(NSUB, NCH, C)
    dst3 = edge_index[1].reshape(NSUB, NCH, C)
    idx4 = jnp.stack([src3, dst3], axis=2)
    z64 = jnp.zeros((NP, DG), jnp.float32)
    z16 = jnp.zeros((NP, DDEN), jnp.float32)
    acc, den = _sc_edge_aggregate(feat2, el, er, idx4, z64, z16)
    return _tc_finalize(acc, den, bias)


# speedup vs baseline: 36.2553x; 1.0653x over previous
import dataclasses
import functools

import jax
import jax.numpy as jnp
from jax import lax
from jax.experimental import pallas as pl
from jax.experimental.pallas import tpu as pltpu
from jax.experimental.pallas import tpu_sc as plsc

N = 10000
NP = 10240
E = 320000
D = 128
DG = 64
DDEN = 16
NSUB = 16
EPT = E // NSUB
C = 96
EPTP = 20160
NCH = EPTP // C
RPT = NP // NSUB
LANES = 16
NBUF = 6


def _tc_project(x, W, attn_l, attn_r):
    def body(x_ref, w_ref, al_ref, ar_ref, feat2_ref, el_ref, er_ref):
        feat = jnp.dot(x_ref[...], w_ref[...],
                       preferred_element_type=jnp.float32)
        feat2_ref[0, :N, :] = feat[:, :DG]
        feat2_ref[1, :N, :] = feat[:, DG:]
        el_ref[pl.ds(0, N)] = jnp.sum(feat * al_ref[...][None, :], axis=1)
        er_ref[pl.ds(0, N)] = jnp.sum(feat * ar_ref[...][None, :], axis=1)
        zpad = jnp.zeros((NP - N, DG), jnp.float32)
        feat2_ref[0, pl.ds(N, NP - N), :] = zpad
        feat2_ref[1, pl.ds(N, NP - N), :] = zpad
        el_ref[pl.ds(N, NP - N)] = jnp.zeros((NP - N,), jnp.float32)
        er_ref[pl.ds(N, NP - N)] = jnp.zeros((NP - N,), jnp.float32)

    return pl.pallas_call(
        body,
        out_shape=(
            jax.ShapeDtypeStruct((2, NP, DG), jnp.float32),
            jax.ShapeDtypeStruct((NP,), jnp.float32),
            jax.ShapeDtypeStruct((NP,), jnp.float32),
        ),
    )(x, W, attn_l, attn_r)


def _sc_edge_aggregate(feat2, el, er, idx4, z64, z16):
    mesh = plsc.VectorSubcoreMesh(core_axis_name="c", subcore_axis_name="s")
    cp = pltpu.CompilerParams()
    if "needs_layout_passes" in pltpu.CompilerParams.__dataclass_fields__:
        cp = dataclasses.replace(cp, needs_layout_passes=False)
    if "use_tc_tiling_on_sc" in pltpu.CompilerParams.__dataclass_fields__:
        cp = dataclasses.replace(cp, use_tc_tiling_on_sc=False)

    @functools.partial(
        pl.kernel,
        compiler_params=cp,
        out_type=(
            jax.ShapeDtypeStruct((2, NP, DG), jnp.float32),
            jax.ShapeDtypeStruct((2, NP, DDEN), jnp.float32),
        ),
        mesh=mesh,
        scratch_types=(
            [
                pltpu.VMEM((NP,), jnp.float32),
                pltpu.VMEM((NP,), jnp.float32),
            ]
            + [pltpu.VMEM((2, C), jnp.int32)] * NBUF
            + [pltpu.VMEM((C, DG), jnp.float32)] * NBUF
            + [pltpu.VMEM((C, DDEN), jnp.float32)] * NBUF
            + [
                pltpu.VMEM_SHARED((NP, DG), jnp.float32),
                pltpu.VMEM_SHARED((NP, DDEN), jnp.float32),
            ]
            + [pltpu.SemaphoreType.DMA] * (3 * NBUF + 1)
        ),
    )
    def k(feat2_hbm, el_hbm, er_hbm, idx_hbm, z64_hbm, z16_hbm,
          acc_out, den_out,
          el_v, er_v, i0, i1, i2, i3, i4, i5, r0, r1, r2, r3, r4, r5,
          w0, w1, w2, w3, w4, w5, acc_sh, den_sh,
          a0, a1, a2, a3, a4, a5, g0, g1, g2, g3, g4, g5,
          s0, s1, s2, s3, s4, s5, stg):
        cid = lax.axis_index("c")
        sid = lax.axis_index("s")
        idxb = [i0, i1, i2, i3, i4, i5]
        rows = [r0, r1, r2, r3, r4, r5]
        wden = [w0, w1, w2, w3, w4, w5]
        isem = [a0, a1, a2, a3, a4, a5]
        gsem = [g0, g1, g2, g3, g4, g5]
        ssem = [s0, s1, s2, s3, s4, s5]
        tile = pl.ds(sid * RPT, RPT)

        copies = [
            (z64_hbm.at[tile], acc_sh.at[tile]),
            (z16_hbm.at[tile], den_sh.at[tile]),
            (el_hbm, el_v),
            (er_hbm, er_v),
        ]
        descs = [pltpu.async_copy(a, b, stg) for a, b in copies]
        for d in descs:
            d.wait()
        plsc.subcore_barrier()

        def idx_start(m, b):
            pltpu.async_copy(idx_hbm.at[sid, m], idxb[b], isem[b])

        def idx_wait(m, b):
            pltpu.make_async_copy(idx_hbm.at[sid, m], idxb[b], isem[b]).wait()

        def gather_start(b):
            pltpu.async_copy(feat2_hbm.at[cid].at[idxb[b].at[0]],
                             rows[b], gsem[b])

        def gather_wait(b):
            pltpu.make_async_copy(feat2_hbm.at[cid].at[idxb[b].at[0]],
                                  rows[b], gsem[b]).wait()

        def scat_start(b):
            pltpu.async_copy(rows[b], acc_sh.at[idxb[b].at[1]], ssem[b],
                             add=True)
            pltpu.async_copy(wden[b], den_sh.at[idxb[b].at[1]], ssem[b],
                             add=True)

        def scat_wait(b):
            pltpu.make_async_copy(rows[b], acc_sh.at[idxb[b].at[1]],
                                  ssem[b]).wait()
            pltpu.make_async_copy(wden[b], den_sh.at[idxb[b].at[1]],
                                  ssem[b]).wait()

        def process(b):
            @plsc.parallel_loop(0, C, step=LANES)
            def _grp(g):
                si = idxb[b][0, pl.ds(g, LANES)]
                di = idxb[b][1, pl.ds(g, LANES)]
                e = plsc.load_gather(el_v, [si]) + plsc.load_gather(er_v, [di])
                e = jnp.where(e >= 0.0, e, 0.2 * e)
                wv = jnp.exp(e)
                for rr in range(LANES):
                    wb = jnp.broadcast_to(wv[rr], (LANES,))
                    wden[b][g + rr, :] = wb
                    for q in range(DG // LANES):
                        sl = pl.ds(q * LANES, LANES)
                        rows[b][g + rr, sl] = rows[b][g + rr, sl] * wb

        idx_start(0, 0)
        idx_start(1, 1)
        idx_start(2, 2)
        idx_wait(0, 0)
        gather_start(0)
        idx_wait(1, 1)
        gather_start(1)
        for s in range(NBUF):
            if s >= 3:
                scat_wait(s - 3)
            idx_start(s + 3, (s + 3) % NBUF)
            idx_wait(s + 2, (s + 2) % NBUF)
            gather_start((s + 2) % NBUF)
            gather_wait(s)
            process(s)
            scat_start(s)

        @pl.loop(NBUF, NCH, step=NBUF)
        def _steady(j):
            for off in range(NBUF):
                m = j + off
                bb3 = (off + 3) % NBUF
                bb2 = (off + 2) % NBUF
                scat_wait(bb3)

                @pl.when(m + 3 < NCH)
                def _():
                    idx_start(m + 3, bb3)

                @pl.when(m + 2 < NCH)
                def _():
                    idx_wait(m + 2, bb2)
                    gather_start(bb2)

                gather_wait(off)
                process(off)
                scat_start(off)

        scat_wait((NCH - 3) % NBUF)
        scat_wait((NCH - 2) % NBUF)
        scat_wait((NCH - 1) % NBUF)

        plsc.subcore_barrier()
        pltpu.sync_copy(acc_sh.at[tile], acc_out.at[cid, tile])
        pltpu.sync_copy(den_sh.at[tile], den_out.at[cid, tile])

    return k(feat2, el, er, idx4, z64, z16)


def _tc_finalize(acc, den, bias):
    def body(a_ref, d_ref, b_ref, o_ref):
        num = jnp.concatenate([a_ref[0, :N, :], a_ref[1, :N, :]], axis=1)
        d = d_ref[0, :N, 0:1]
        o_ref[...] = jnp.where(d > 0.0, num / d, 0.0) + b_ref[...][None, :]

    return pl.pallas_call(
        body,
        out_shape=jax.ShapeDtypeStruct((N, D), jnp.float32),
    )(acc, den, bias)


def kernel(x, edge_index, W, attn_l, attn_r, bias):
    feat2, el, er = _tc_project(x, W, attn_l, attn_r)
    ei3 = edge_index.reshape(2, NSUB, EPT)
    pad = jnp.full((2, NSUB, EPTP - EPT), N, jnp.int32)
    ei4 = jnp.concatenate([ei3, pad], axis=2)
    idx4 = ei4.reshape(2, NSUB, NCH, C).transpose(1, 2, 0, 3)
    z64 = jnp.zeros((NP, DG), jnp.float32)
    z16 = jnp.zeros((NP, DDEN), jnp.float32)
    acc, den = _sc_edge_aggregate(feat2, el, er, idx4, z64, z16)
    return _tc_finalize(acc, den, bias)


# speedup vs baseline: 36.6075x; 1.0097x over previous
import dataclasses
import functools

import jax
import jax.numpy as jnp
from jax import lax
from jax.experimental import pallas as pl
from jax.experimental.pallas import tpu as pltpu
from jax.experimental.pallas import tpu_sc as plsc

N = 10000
NP = 10240
E = 320000
D = 128
DG = 64
DDEN = 16
NSUB = 16
EPT = E // NSUB
C = 96
EPTP = 20160
NCH = EPTP // C
RPT = NP // NSUB
LANES = 16
NBUF = 6


def _tc_project(x, W, attn_l, attn_r):
    def body(x_ref, w_ref, al_ref, ar_ref, feat2_ref, el_ref, er_ref):
        feat = jnp.dot(x_ref[...], w_ref[...],
                       preferred_element_type=jnp.float32)
        feat2_ref[0, :N, :] = feat[:, :DG]
        feat2_ref[1, :N, :] = feat[:, DG:]
        el_ref[pl.ds(0, N)] = jnp.sum(feat * al_ref[...][None, :], axis=1)
        er_ref[pl.ds(0, N)] = jnp.sum(feat * ar_ref[...][None, :], axis=1)
        zpad = jnp.zeros((NP - N, DG), jnp.float32)
        feat2_ref[0, pl.ds(N, NP - N), :] = zpad
        feat2_ref[1, pl.ds(N, NP - N), :] = zpad
        el_ref[pl.ds(N, NP - N)] = jnp.zeros((NP - N,), jnp.float32)
        er_ref[pl.ds(N, NP - N)] = jnp.zeros((NP - N,), jnp.float32)

    return pl.pallas_call(
        body,
        out_shape=(
            jax.ShapeDtypeStruct((2, NP, DG), jnp.float32),
            jax.ShapeDtypeStruct((NP,), jnp.float32),
            jax.ShapeDtypeStruct((NP,), jnp.float32),
        ),
    )(x, W, attn_l, attn_r)


FB = 80


def _sc_edge_aggregate(feat2, el, er, idx4, z64, z16, bias):
    mesh = plsc.VectorSubcoreMesh(core_axis_name="c", subcore_axis_name="s")
    cp = pltpu.CompilerParams()
    if "needs_layout_passes" in pltpu.CompilerParams.__dataclass_fields__:
        cp = dataclasses.replace(cp, needs_layout_passes=False)
    if "use_tc_tiling_on_sc" in pltpu.CompilerParams.__dataclass_fields__:
        cp = dataclasses.replace(cp, use_tc_tiling_on_sc=False)

    @functools.partial(
        pl.kernel,
        compiler_params=cp,
        out_type=jax.ShapeDtypeStruct((2, NP, DG), jnp.float32),
        mesh=mesh,
        scratch_types=(
            [
                pltpu.VMEM((NP,), jnp.float32),
                pltpu.VMEM((NP,), jnp.float32),
                pltpu.VMEM((D,), jnp.float32),
            ]
            + [pltpu.VMEM((2, C), jnp.int32)] * NBUF
            + [pltpu.VMEM((C, DG), jnp.float32)] * NBUF
            + [pltpu.VMEM((C, DDEN), jnp.float32)] * NBUF
            + [
                pltpu.VMEM_SHARED((NP, DG), jnp.float32),
                pltpu.VMEM_SHARED((NP, DDEN), jnp.float32),
            ]
            + [pltpu.SemaphoreType.DMA] * (3 * NBUF + 1)
        ),
    )
    def k(feat2_hbm, el_hbm, er_hbm, idx_hbm, z64_hbm, z16_hbm, bias_hbm,
          out_hbm,
          el_v, er_v, bias_v, i0, i1, i2, i3, i4, i5, r0, r1, r2, r3, r4, r5,
          w0, w1, w2, w3, w4, w5, acc_sh, den_sh,
          a0, a1, a2, a3, a4, a5, g0, g1, g2, g3, g4, g5,
          s0, s1, s2, s3, s4, s5, stg):
        cid = lax.axis_index("c")
        sid = lax.axis_index("s")
        idxb = [i0, i1, i2, i3, i4, i5]
        rows = [r0, r1, r2, r3, r4, r5]
        wden = [w0, w1, w2, w3, w4, w5]
        isem = [a0, a1, a2, a3, a4, a5]
        gsem = [g0, g1, g2, g3, g4, g5]
        ssem = [s0, s1, s2, s3, s4, s5]
        tile = pl.ds(sid * RPT, RPT)

        copies = [
            (z64_hbm.at[tile], acc_sh.at[tile]),
            (z16_hbm.at[tile], den_sh.at[tile]),
            (el_hbm, el_v),
            (er_hbm, er_v),
            (bias_hbm, bias_v),
        ]
        descs = [pltpu.async_copy(a, b, stg) for a, b in copies]
        for d in descs:
            d.wait()
        plsc.subcore_barrier()

        def idx_start(m, b):
            pltpu.async_copy(idx_hbm.at[sid, m], idxb[b], isem[b])

        def idx_wait(m, b):
            pltpu.make_async_copy(idx_hbm.at[sid, m], idxb[b], isem[b]).wait()

        def gather_start(b):
            pltpu.async_copy(feat2_hbm.at[cid].at[idxb[b].at[0]],
                             rows[b], gsem[b])

        def gather_wait(b):
            pltpu.make_async_copy(feat2_hbm.at[cid].at[idxb[b].at[0]],
                                  rows[b], gsem[b]).wait()

        def scat_start(b):
            pltpu.async_copy(rows[b], acc_sh.at[idxb[b].at[1]], ssem[b],
                             add=True)
            pltpu.async_copy(wden[b], den_sh.at[idxb[b].at[1]], ssem[b],
                             add=True)

        def scat_wait(b):
            pltpu.make_async_copy(rows[b], acc_sh.at[idxb[b].at[1]],
                                  ssem[b]).wait()
            pltpu.make_async_copy(wden[b], den_sh.at[idxb[b].at[1]],
                                  ssem[b]).wait()

        def process(b):
            @plsc.parallel_loop(0, C, step=LANES)
            def _grp(g):
                si = idxb[b][0, pl.ds(g, LANES)]
                di = idxb[b][1, pl.ds(g, LANES)]
                e = plsc.load_gather(el_v, [si]) + plsc.load_gather(er_v, [di])
                e = jnp.where(e >= 0.0, e, 0.2 * e)
                wv = jnp.exp(e)
                for rr in range(LANES):
                    wb = jnp.broadcast_to(wv[rr], (LANES,))
                    wden[b][g + rr, :] = wb
                    for q in range(DG // LANES):
                        sl = pl.ds(q * LANES, LANES)
                        rows[b][g + rr, sl] = rows[b][g + rr, sl] * wb

        idx_start(0, 0)
        idx_start(1, 1)
        idx_start(2, 2)
        idx_wait(0, 0)
        gather_start(0)
        idx_wait(1, 1)
        gather_start(1)
        for s in range(NBUF):
            if s >= 3:
                scat_wait(s - 3)
            idx_start(s + 3, (s + 3) % NBUF)
            idx_wait(s + 2, (s + 2) % NBUF)
            gather_start((s + 2) % NBUF)
            gather_wait(s)
            process(s)
            scat_start(s)

        @pl.loop(NBUF, NCH, step=NBUF)
        def _steady(j):
            for off in range(NBUF):
                m = j + off
                bb3 = (off + 3) % NBUF
                bb2 = (off + 2) % NBUF
                scat_wait(bb3)

                @pl.when(m + 3 < NCH)
                def _():
                    idx_start(m + 3, bb3)

                @pl.when(m + 2 < NCH)
                def _():
                    idx_wait(m + 2, bb2)
                    gather_start(bb2)

                gather_wait(off)
                process(off)
                scat_start(off)

        scat_wait((NCH - 3) % NBUF)
        scat_wait((NCH - 2) % NBUF)
        scat_wait((NCH - 1) % NBUF)

        plsc.subcore_barrier()

        bias_regs = [bias_v[pl.ds(cid * DG + q * LANES, LANES)]
                     for q in range(DG // LANES)]

        @pl.loop(0, RPT, step=FB)
        def _fin(f):
            rb = sid * RPT + f
            pltpu.sync_copy(acc_sh.at[pl.ds(rb, FB)], r0.at[pl.ds(0, FB)])
            pltpu.sync_copy(den_sh.at[pl.ds(rb, FB)], w0.at[pl.ds(0, FB)])

            @pl.loop(0, FB)
            def _r(r):
                db = w0[r, pl.ds(0, LANES)]
                ok = db > 0.0
                for q in range(DG // LANES):
                    sl = pl.ds(q * LANES, LANES)
                    v = r0[r, sl]
                    r0[r, sl] = jnp.where(ok, v / db, 0.0) + bias_regs[q]

            pltpu.sync_copy(r0.at[pl.ds(0, FB)], out_hbm.at[cid, pl.ds(rb, FB)])

    return k(feat2, el, er, idx4, z64, z16, bias)


def kernel(x, edge_index, W, attn_l, attn_r, bias):
    feat2, el, er = _tc_project(x, W, attn_l, attn_r)
    ei3 = edge_index.reshape(2, NSUB, EPT)
    pad = jnp.full((2, NSUB, EPTP - EPT), N, jnp.int32)
    ei4 = jnp.concatenate([ei3, pad], axis=2)
    idx4 = ei4.reshape(2, NSUB, NCH, C).transpose(1, 2, 0, 3)
    z64 = jnp.zeros((NP, DG), jnp.float32)
    z16 = jnp.zeros((NP, DDEN), jnp.float32)
    halves = _sc_edge_aggregate(feat2, el, er, idx4, z64, z16, bias)
    return jnp.concatenate([halves[0, :N], halves[1, :N]], axis=1)
